# Initial kernel scaffold; baseline (speedup 1.0000x reference)
import jax
import jax.numpy as jnp
from jax.experimental import pallas as pl


def kernel(node_features, edge_index, status, W1, b1, W2, b2, Wa1, ba1, Wa2, ba2, Wc1, bc1, Wc2, bc2, Wc3, bc3):
    raise NotImplementedError("write your pallas kernel here")


# speedup vs baseline: 3.0676x; 3.0676x over previous
import functools

import jax
import jax.numpy as jnp
from jax import lax
from jax.experimental import pallas as pl
from jax.experimental.pallas import tpu as pltpu
from jax.experimental.pallas import tpu_sc as plsc

NC = 2
NS = 16
L = 16
W = 128

_mesh = lambda: plsc.VectorSubcoreMesh(core_axis_name="c", subcore_axis_name="s")


def _make_deg(np_, e_pad):
    sh = e_pad // (NC * NS)
    ch = sh // 2
    nwin = ch // W

    sh2 = e_pad // NS
    ch2 = sh2 // 4
    nwin2 = ch2 // W
    per = np_ // NS

    @functools.partial(
        pl.kernel,
        out_type=jax.ShapeDtypeStruct((np_, L), jnp.float32),
        mesh=_mesh(),
        compiler_params=pltpu.CompilerParams(needs_layout_passes=False),
        scratch_types=[
            pltpu.VMEM_SHARED((np_, L), jnp.float32),
            pltpu.VMEM((ch2,), jnp.int32),
            pltpu.VMEM((1, W), jnp.int32),
            pltpu.VMEM((W, L), jnp.float32),
            pltpu.SemaphoreType.DMA,
        ],
    )
    def deg(dst_hbm, z_hbm, ones_hbm, out_hbm, acc, dstb, widx, ones, sem):
        c = lax.axis_index("c")
        s = lax.axis_index("s")

        @pl.when(c == 0)
        def _():
            pltpu.sync_copy(ones_hbm, ones)
            pltpu.async_copy(z_hbm.at[pl.ds(s * per, per)],
                             acc.at[pl.ds(s * per, per)], sem).wait()
            plsc.subcore_barrier()

            for k in range(4):
                off = s * sh2 + k * ch2
                pltpu.sync_copy(dst_hbm.at[pl.ds(off, ch2)], dstb)

                def win(i, _):
                    w0 = i * W
                    for j in range(W // L):
                        widx[0, pl.ds(j * L, L)] = dstb[pl.ds(w0 + j * L, L)]
                    pltpu.async_copy(ones, acc.at[widx.at[0]], sem,
                                     add=True).wait()
                    return 0
                lax.fori_loop(0, nwin2, win, 0)
            plsc.subcore_barrier()

            pltpu.async_copy(acc.at[pl.ds(s * per, per)],
                             out_hbm.at[pl.ds(s * per, per)], sem).wait()

    return deg


def _make_scat(np_, e_pad):
    qp = np_ // 4
    accr = qp + L
    sh = e_pad // NS
    ch = sh // 16
    steps = ch // L
    ws = 64
    maxsel = ch + ws + L
    zper = qp // NS

    @functools.partial(
        pl.kernel,
        out_type=jax.ShapeDtypeStruct((np_, 128), jnp.float32),
        mesh=_mesh(),
        compiler_params=pltpu.CompilerParams(needs_layout_passes=False),
        scratch_types=[
            pltpu.VMEM_SHARED((accr, 128), jnp.float32),
            pltpu.VMEM((ch,), jnp.int32),
            pltpu.VMEM((ch,), jnp.int32),
            pltpu.VMEM((maxsel,), jnp.int32),
            pltpu.VMEM((maxsel,), jnp.int32),
            pltpu.VMEM((1, ws), jnp.int32),
            pltpu.VMEM((1, ws), jnp.int32),
            pltpu.VMEM((ws, 128), jnp.float32),
            pltpu.SemaphoreType.DMA,
        ],
    )
    def scat(src_hbm, dst_hbm, y_hbm, z_hbm, out_hbm,
             acc, srcb, dstb, selsrc, selloc, gidx, sidx, rowbuf, sem):
        c = lax.axis_index("c")
        s = lax.axis_index("s")

        padsrc = jnp.full((L,), s * 64, jnp.int32)
        padloc = jnp.full((L,), qp + s, jnp.int32)

        for p in range(2):
            base = (2 * p + c) * qp

            pltpu.async_copy(z_hbm, acc.at[pl.ds(s * zper, zper)],
                             sem).wait()
            plsc.subcore_barrier()

            for k in range(16):
                off = s * sh + k * ch
                pltpu.sync_copy(src_hbm.at[pl.ds(off, ch)], srcb)
                pltpu.sync_copy(dst_hbm.at[pl.ds(off, ch)], dstb)

                ones_v = jnp.ones((L,), jnp.int32)

                def scan(i, cntv):
                    o = i * L
                    srcv = srcb[pl.ds(o, L)]
                    dstv = dstb[pl.ds(o, L)]
                    m = (dstv >= base) & (dstv < base + qp)
                    pos = cntv + plsc.cumsum(ones_v, mask=m) - 1
                    plsc.store_scatter(selsrc, [pos], srcv, mask=m)
                    plsc.store_scatter(selloc, [pos], dstv - base, mask=m)
                    return cntv + plsc.all_reduce_population_count(m)
                cntv = lax.fori_loop(0, steps, scan,
                                     jnp.zeros((L,), jnp.int32))
                cnt = cntv[0]

                selsrc[pl.ds(cnt, L)] = padsrc
                selloc[pl.ds(cnt, L)] = padloc
                cnt = ((cnt + L - 1) // L) * L
                for j in range(ws // L - 1):
                    selsrc[pl.ds(cnt + j * L, L)] = padsrc
                    selloc[pl.ds(cnt + j * L, L)] = padloc
                cnt = ((cnt + ws - 1) // ws) * ws

                def win(i, _):
                    w0 = i * ws
                    for j in range(ws // L):
                        gidx[0, pl.ds(j * L, L)] = selsrc[pl.ds(w0 + j * L, L)]
                        sidx[0, pl.ds(j * L, L)] = selloc[pl.ds(w0 + j * L, L)]
                    pltpu.async_copy(y_hbm.at[gidx.at[0]], rowbuf, sem).wait()
                    pltpu.async_copy(rowbuf, acc.at[sidx.at[0]], sem,
                                     add=True).wait()
                    return 0
                lax.fori_loop(0, cnt // ws, win, 0)

            plsc.subcore_barrier()

            pltpu.async_copy(acc.at[pl.ds(s * zper, zper)],
                             out_hbm.at[pl.ds(base + s * zper, zper)],
                             sem).wait()
            plsc.subcore_barrier()

    return scat


def _tc_y1(xpad, w1p, degp, np_, blk):
    grid = np_ // blk

    def body(x_ref, w_ref, d_ref, y_ref, dis_ref):
        hist = d_ref[...]
        dis = lax.rsqrt(hist + 1.0)
        xw = jnp.dot(x_ref[...], w_ref[...], preferred_element_type=jnp.float32)
        y_ref[...] = xw * dis
        dis_ref[...] = dis

    return pl.pallas_call(
        body,
        grid=(grid,),
        in_specs=[
            pl.BlockSpec((blk, xpad.shape[1]), lambda i: (i, 0)),
            pl.BlockSpec(w1p.shape, lambda i: (0, 0)),
            pl.BlockSpec((blk, 1), lambda i: (i, 0)),
        ],
        out_specs=[
            pl.BlockSpec((blk, 128), lambda i: (i, 0)),
            pl.BlockSpec((blk, 1), lambda i: (i, 0)),
        ],
        out_shape=[
            jax.ShapeDtypeStruct((np_, 128), jnp.float32),
            jax.ShapeDtypeStruct((np_, 1), jnp.float32),
        ],
    )(xpad, w1p, degp)


def _tc_mid(acc1, y1, dis, b1, w2, np_, blk):
    grid = np_ // blk

    def body(a_ref, y_ref, d_ref, b_ref, w_ref, o_ref):
        x1 = jnp.maximum(d_ref[...] * (a_ref[...] + y_ref[...]) + b_ref[...],
                         0.0)
        o_ref[...] = d_ref[...] * jnp.dot(
            x1, w_ref[...], preferred_element_type=jnp.float32)

    return pl.pallas_call(
        body,
        grid=(grid,),
        in_specs=[
            pl.BlockSpec((blk, 128), lambda i: (i, 0)),
            pl.BlockSpec((blk, 128), lambda i: (i, 0)),
            pl.BlockSpec((blk, 1), lambda i: (i, 0)),
            pl.BlockSpec((1, 128), lambda i: (0, 0)),
            pl.BlockSpec((128, 128), lambda i: (0, 0)),
        ],
        out_specs=pl.BlockSpec((blk, 128), lambda i: (i, 0)),
        out_shape=jax.ShapeDtypeStruct((np_, 128), jnp.float32),
    )(acc1, y1, dis, b1, w2)


def _tc_x2(acc2, y2, dis, b2, status2, np_, n_real, blk):
    grid = np_ // blk

    def body(a_ref, y_ref, d_ref, b_ref, st_ref, x2_ref, gs_ref, c1_ref):
        i = pl.program_id(0)
        x2 = jnp.maximum(d_ref[...] * (a_ref[...] + y_ref[...]) + b_ref[...],
                         0.0)
        x2_ref[...] = x2

        @pl.when(i == 0)
        def _():
            gs_ref[...] = jnp.zeros_like(gs_ref)
            c1_ref[...] = jnp.zeros_like(c1_ref)

        row = i * blk + lax.broadcasted_iota(jnp.int32, (blk, 1), 0)
        real = row < n_real
        gs_ref[...] += jnp.sum(jnp.where(real, x2, 0.0), axis=0, keepdims=True)
        c1_ref[...] += jnp.sum((st_ref[...] == 1).astype(jnp.float32),
                               keepdims=True)

    return pl.pallas_call(
        body,
        grid=(grid,),
        in_specs=[
            pl.BlockSpec((blk, 128), lambda i: (i, 0)),
            pl.BlockSpec((blk, 128), lambda i: (i, 0)),
            pl.BlockSpec((blk, 1), lambda i: (i, 0)),
            pl.BlockSpec((1, 128), lambda i: (0, 0)),
            pl.BlockSpec((blk, 1), lambda i: (i, 0)),
        ],
        out_specs=[
            pl.BlockSpec((blk, 128), lambda i: (i, 0)),
            pl.BlockSpec((1, 128), lambda i: (0, 0)),
            pl.BlockSpec((1, 1), lambda i: (0, 0)),
        ],
        out_shape=[
            jax.ShapeDtypeStruct((np_, 128), jnp.float32),
            jax.ShapeDtypeStruct((1, 128), jnp.float32),
            jax.ShapeDtypeStruct((1, 1), jnp.float32),
        ],
    )(acc2, y2, dis, b2, status2)


def _tc_heads(x2, gsum, cnt1, status2, wa1, ba1, wa2, ba2,
              wc1, bc1, wc2, bc2, wc3, bc3, np_, n_real, blk):
    grid = np_ // blk
    hid = wa1.shape[1]

    def body(x_ref, gs_ref, c1_ref, st_ref, wa1_ref, ba1_ref, wa2_ref,
             ba2_ref, wc1_ref, bc1_ref, wc2_ref, bc2_ref, wc3_ref, bc3_ref,
             lg_ref, sv_ref):
        g = gs_ref[...] * (1.0 / n_real)
        c0 = jnp.dot(g, wa1_ref[hid:, :],
                     preferred_element_type=jnp.float32) + ba1_ref[...]
        h = jnp.maximum(
            jnp.dot(x_ref[...], wa1_ref[:hid, :],
                    preferred_element_type=jnp.float32) + c0, 0.0)
        lg = jnp.dot(h, wa2_ref[...],
                     preferred_element_type=jnp.float32) + ba2_ref[...]
        target = jnp.where(c1_ref[0, 0] > 0.0, 1, 0)
        sel = st_ref[...] == target
        lg_ref[...] = jnp.where(sel, lg, -1000000000.0)

        @pl.when(pl.program_id(0) == 0)
        def _():
            pooled = jnp.concatenate([g, g], axis=1)
            cc = jnp.maximum(jnp.dot(pooled, wc1_ref[...],
                                     preferred_element_type=jnp.float32)
                             + bc1_ref[...], 0.0)
            cc = jnp.maximum(jnp.dot(cc, wc2_ref[...],
                                     preferred_element_type=jnp.float32)
                             + bc2_ref[...], 0.0)
            sv_ref[...] = jnp.dot(cc, wc3_ref[...],
                                  preferred_element_type=jnp.float32) \
                + bc3_ref[...]

    full = lambda a: pl.BlockSpec(a.shape, lambda i: tuple(0 for _ in a.shape))
    return pl.pallas_call(
        body,
        grid=(grid,),
        in_specs=[
            pl.BlockSpec((blk, 128), lambda i: (i, 0)),
            full(gsum), full(cnt1),
            pl.BlockSpec((blk, 1), lambda i: (i, 0)),
            full(wa1), full(ba1), full(wa2), full(ba2),
            full(wc1), full(bc1), full(wc2), full(bc2), full(wc3), full(bc3),
        ],
        out_specs=[
            pl.BlockSpec((blk, 1), lambda i: (i, 0)),
            pl.BlockSpec((1, 1), lambda i: (0, 0)),
        ],
        out_shape=[
            jax.ShapeDtypeStruct((np_, 1), jnp.float32),
            jax.ShapeDtypeStruct((1, 1), jnp.float32),
        ],
    )(x2, gsum, cnt1, status2, wa1, ba1, wa2, ba2,
      wc1, bc1, wc2, bc2, wc3, bc3)


def kernel(node_features, edge_index, status, W1, b1, W2, b2, Wa1, ba1,
           Wa2, ba2, Wc1, bc1, Wc2, bc2, Wc3, bc3):
    n, feat = node_features.shape
    e = edge_index.shape[1]

    qp = -(-n // (4 * W)) * W
    np_ = 4 * qp
    blk = np_ // 32
    assert np_ % 32 == 0 and blk % 8 == 0

    unit = NC * NS * W * 2 * 2
    e_pad = -(-e // unit) * unit
    padn = e_pad - e

    src = edge_index[0]
    dst = edge_index[1]
    idx = jnp.arange(padn, dtype=jnp.int32)
    srcp = jnp.concatenate([src, idx % n])
    dstp = jnp.concatenate([dst, n + (idx % L)])

    fpad = 16 - feat
    xpad = jnp.pad(node_features, ((0, np_ - n), (0, fpad)))
    w1p = jnp.pad(W1, ((0, fpad), (0, 0)))

    status2 = jnp.pad(status.reshape(n, 1), ((0, np_ - n), (0, 0)),
                      constant_values=2)
    b1r = b1.reshape(1, -1)
    b2r = b2.reshape(1, -1)
    ba1r = ba1.reshape(1, -1)
    ba2r = ba2.reshape(1, -1)
    bc1r = bc1.reshape(1, -1)
    bc2r = bc2.reshape(1, -1)
    bc3r = bc3.reshape(1, -1)


    hist = jax.ops.segment_sum(jnp.ones((e,), jnp.float32), dst,
                               num_segments=np_)
    degp = hist.reshape(np_, 1)
    y1, dis = _tc_y1(xpad, w1p, degp, np_, blk)
    acc1 = jax.ops.segment_sum(y1[src], dst, num_segments=np_)
    y2 = _tc_mid(acc1, y1, dis, b1r, W2, np_, blk)
    acc2 = jax.ops.segment_sum(y2[src], dst, num_segments=np_)
    x2, gsum, cnt1 = _tc_x2(acc2, y2, dis, b2r, status2, np_, n, blk)
    lg, sv = _tc_heads(x2, gsum, cnt1, status2, Wa1, ba1r, Wa2, ba2r,
                       Wc1, bc1r, Wc2, bc2r, Wc3, bc3r, np_, n, blk)
    return lg[:n].reshape(n), sv.reshape(())


# speedup vs baseline: 3.0685x; 1.0003x over previous
import jax
import jax.numpy as jnp
from jax import lax
from jax.experimental import pallas as pl

NC = 2
NS = 16
L = 16
W = 128


def _tc_y1(xpad, w1p, degp, np_, blk):
    grid = np_ // blk

    def body(x_ref, w_ref, d_ref, y_ref, dis_ref):
        hist = d_ref[...]
        dis = lax.rsqrt(hist + 1.0)
        xw = jnp.dot(x_ref[...], w_ref[...], preferred_element_type=jnp.float32)
        y_ref[...] = xw * dis
        dis_ref[...] = dis

    return pl.pallas_call(
        body,
        grid=(grid,),
        in_specs=[
            pl.BlockSpec((blk, xpad.shape[1]), lambda i: (i, 0)),
            pl.BlockSpec(w1p.shape, lambda i: (0, 0)),
            pl.BlockSpec((blk, 1), lambda i: (i, 0)),
        ],
        out_specs=[
            pl.BlockSpec((blk, 128), lambda i: (i, 0)),
            pl.BlockSpec((blk, 1), lambda i: (i, 0)),
        ],
        out_shape=[
            jax.ShapeDtypeStruct((np_, 128), jnp.float32),
            jax.ShapeDtypeStruct((np_, 1), jnp.float32),
        ],
    )(xpad, w1p, degp)


def _tc_mid(acc1, y1, dis, b1, w2, np_, blk):
    grid = np_ // blk

    def body(a_ref, y_ref, d_ref, b_ref, w_ref, o_ref):
        x1 = jnp.maximum(d_ref[...] * (a_ref[...] + y_ref[...]) + b_ref[...],
                         0.0)
        o_ref[...] = d_ref[...] * jnp.dot(
            x1, w_ref[...], preferred_element_type=jnp.float32)

    return pl.pallas_call(
        body,
        grid=(grid,),
        in_specs=[
            pl.BlockSpec((blk, 128), lambda i: (i, 0)),
            pl.BlockSpec((blk, 128), lambda i: (i, 0)),
            pl.BlockSpec((blk, 1), lambda i: (i, 0)),
            pl.BlockSpec((1, 128), lambda i: (0, 0)),
            pl.BlockSpec((128, 128), lambda i: (0, 0)),
        ],
        out_specs=pl.BlockSpec((blk, 128), lambda i: (i, 0)),
        out_shape=jax.ShapeDtypeStruct((np_, 128), jnp.float32),
    )(acc1, y1, dis, b1, w2)


def _tc_x2(acc2, y2, dis, b2, status2, np_, n_real, blk):
    grid = np_ // blk

    def body(a_ref, y_ref, d_ref, b_ref, st_ref, x2_ref, gs_ref, c1_ref):
        i = pl.program_id(0)
        x2 = jnp.maximum(d_ref[...] * (a_ref[...] + y_ref[...]) + b_ref[...],
                         0.0)
        x2_ref[...] = x2

        @pl.when(i == 0)
        def _():
            gs_ref[...] = jnp.zeros_like(gs_ref)
            c1_ref[...] = jnp.zeros_like(c1_ref)

        row = i * blk + lax.broadcasted_iota(jnp.int32, (blk, 1), 0)
        real = row < n_real
        gs_ref[...] += jnp.sum(jnp.where(real, x2, 0.0), axis=0, keepdims=True)
        c1_ref[...] += jnp.sum((st_ref[...] == 1).astype(jnp.float32),
                               keepdims=True)

    return pl.pallas_call(
        body,
        grid=(grid,),
        in_specs=[
            pl.BlockSpec((blk, 128), lambda i: (i, 0)),
            pl.BlockSpec((blk, 128), lambda i: (i, 0)),
            pl.BlockSpec((blk, 1), lambda i: (i, 0)),
            pl.BlockSpec((1, 128), lambda i: (0, 0)),
            pl.BlockSpec((blk, 1), lambda i: (i, 0)),
        ],
        out_specs=[
            pl.BlockSpec((blk, 128), lambda i: (i, 0)),
            pl.BlockSpec((1, 128), lambda i: (0, 0)),
            pl.BlockSpec((1, 1), lambda i: (0, 0)),
        ],
        out_shape=[
            jax.ShapeDtypeStruct((np_, 128), jnp.float32),
            jax.ShapeDtypeStruct((1, 128), jnp.float32),
            jax.ShapeDtypeStruct((1, 1), jnp.float32),
        ],
    )(acc2, y2, dis, b2, status2)


def _tc_heads(x2, gsum, cnt1, status2, wa1, ba1, wa2, ba2,
              wc1, bc1, wc2, bc2, wc3, bc3, np_, n_real, blk):
    grid = np_ // blk
    hid = wa1.shape[1]

    def body(x_ref, gs_ref, c1_ref, st_ref, wa1_ref, ba1_ref, wa2_ref,
             ba2_ref, wc1_ref, bc1_ref, wc2_ref, bc2_ref, wc3_ref, bc3_ref,
             lg_ref, sv_ref):
        g = gs_ref[...] * (1.0 / n_real)
        c0 = jnp.dot(g, wa1_ref[hid:, :],
                     preferred_element_type=jnp.float32) + ba1_ref[...]
        h = jnp.maximum(
            jnp.dot(x_ref[...], wa1_ref[:hid, :],
                    preferred_element_type=jnp.float32) + c0, 0.0)
        lg = jnp.dot(h, wa2_ref[...],
                     preferred_element_type=jnp.float32) + ba2_ref[...]
        target = jnp.where(c1_ref[0, 0] > 0.0, 1, 0)
        sel = st_ref[...] == target
        lg_ref[...] = jnp.where(sel, lg, -1000000000.0)

        @pl.when(pl.program_id(0) == 0)
        def _():
            pooled = jnp.concatenate([g, g], axis=1)
            cc = jnp.maximum(jnp.dot(pooled, wc1_ref[...],
                                     preferred_element_type=jnp.float32)
                             + bc1_ref[...], 0.0)
            cc = jnp.maximum(jnp.dot(cc, wc2_ref[...],
                                     preferred_element_type=jnp.float32)
                             + bc2_ref[...], 0.0)
            sv_ref[...] = jnp.dot(cc, wc3_ref[...],
                                  preferred_element_type=jnp.float32) \
                + bc3_ref[...]

    full = lambda a: pl.BlockSpec(a.shape, lambda i: tuple(0 for _ in a.shape))
    return pl.pallas_call(
        body,
        grid=(grid,),
        in_specs=[
            pl.BlockSpec((blk, 128), lambda i: (i, 0)),
            full(gsum), full(cnt1),
            pl.BlockSpec((blk, 1), lambda i: (i, 0)),
            full(wa1), full(ba1), full(wa2), full(ba2),
            full(wc1), full(bc1), full(wc2), full(bc2), full(wc3), full(bc3),
        ],
        out_specs=[
            pl.BlockSpec((blk, 1), lambda i: (i, 0)),
            pl.BlockSpec((1, 1), lambda i: (0, 0)),
        ],
        out_shape=[
            jax.ShapeDtypeStruct((np_, 1), jnp.float32),
            jax.ShapeDtypeStruct((1, 1), jnp.float32),
        ],
    )(x2, gsum, cnt1, status2, wa1, ba1, wa2, ba2,
      wc1, bc1, wc2, bc2, wc3, bc3)


def kernel(node_features, edge_index, status, W1, b1, W2, b2, Wa1, ba1,
           Wa2, ba2, Wc1, bc1, Wc2, bc2, Wc3, bc3):
    n, feat = node_features.shape
    e = edge_index.shape[1]

    qp = -(-n // (4 * W)) * W
    np_ = 4 * qp
    blk = np_ // 32
    assert np_ % 32 == 0 and blk % 8 == 0

    unit = NC * NS * W * 2 * 2
    e_pad = -(-e // unit) * unit
    padn = e_pad - e

    src = edge_index[0]
    dst = edge_index[1]
    idx = jnp.arange(padn, dtype=jnp.int32)
    srcp = jnp.concatenate([src, idx % n])
    dstp = jnp.concatenate([dst, n + (idx % L)])

    fpad = 16 - feat
    xpad = jnp.pad(node_features, ((0, np_ - n), (0, fpad)))
    w1p = jnp.pad(W1, ((0, fpad), (0, 0)))

    status2 = jnp.pad(status.reshape(n, 1), ((0, np_ - n), (0, 0)),
                      constant_values=2)
    b1r = b1.reshape(1, -1)
    b2r = b2.reshape(1, -1)
    ba1r = ba1.reshape(1, -1)
    ba2r = ba2.reshape(1, -1)
    bc1r = bc1.reshape(1, -1)
    bc2r = bc2.reshape(1, -1)
    bc3r = bc3.reshape(1, -1)


    hist = jax.ops.segment_sum(jnp.ones((e,), jnp.float32), dst,
                               num_segments=np_)
    degp = hist.reshape(np_, 1)
    y1, dis = _tc_y1(xpad, w1p, degp, np_, blk)
    acc1 = jax.ops.segment_sum(y1[src], dst, num_segments=np_)
    y2 = _tc_mid(acc1, y1, dis, b1r, W2, np_, blk)
    acc2 = jax.ops.segment_sum(y2[src], dst, num_segments=np_)
    x2, gsum, cnt1 = _tc_x2(acc2, y2, dis, b2r, status2, np_, n, blk)
    lg, sv = _tc_heads(x2, gsum, cnt1, status2, Wa1, ba1r, Wa2, ba2r,
                       Wc1, bc1r, Wc2, bc2r, Wc3, bc3r, np_, n, blk)
    return lg[:n].reshape(n), sv.reshape(())
